# baseline (speedup 1.0000x reference)
import functools

import jax
import jax.numpy as jnp
from jax import lax
from jax.experimental import pallas as pl
from jax.experimental.pallas import tpu as pltpu
from jax.experimental.pallas import tpu_sc as plsc

_NC = 2
_NS = 16
_NW = _NC * _NS
_CH = 256
_NB = 2


def _tc_k_body(pos_ref, key_ref, kc_ref, ko_ref):
    del pos_ref, kc_ref
    ko_ref[...] = key_ref[...]


def _sc_v_body(value_ref, vc_ref, rowidx_ref, out_ref,
               buf, newbuf, idxbuf, in_sems, out_sems, sc_sem):
    nrows = vc_ref.shape[0]
    npairs = value_ref.shape[0]
    num = value_ref.shape[1]
    w = lax.axis_index("s") * _NC + lax.axis_index("c")
    rows_w = nrows // _NW
    base = w * rows_w
    groups = rows_w // (_CH * _NB)

    for b in range(_NB):
        pltpu.make_async_copy(
            vc_ref.at[pl.ds(base + b * _CH, _CH)], buf.at[b],
            in_sems.at[b]).start()

    def group(gi, carry):
        g0 = base + gi * (_CH * _NB)
        for b in range(_NB):
            pltpu.make_async_copy(
                vc_ref.at[pl.ds(g0 + b * _CH, _CH)], buf.at[b],
                in_sems.at[b]).wait()
            pltpu.make_async_copy(
                buf.at[b], out_ref.at[pl.ds(g0 + b * _CH, _CH)],
                out_sems.at[b]).start()
        n0 = g0 + _NB * _CH

        @pl.when(gi + 1 < groups)
        def _():
            for b in range(_NB):
                pltpu.make_async_copy(
                    buf.at[b], out_ref.at[pl.ds(g0 + b * _CH, _CH)],
                    out_sems.at[b]).wait()
                pltpu.make_async_copy(
                    vc_ref.at[pl.ds(n0 + b * _CH, _CH)], buf.at[b],
                    in_sems.at[b]).start()

        @pl.when(gi + 1 >= groups)
        def _():
            for b in range(_NB):
                pltpu.make_async_copy(
                    buf.at[b], out_ref.at[pl.ds(g0 + b * _CH, _CH)],
                    out_sems.at[b]).wait()
        return carry

    lax.fori_loop(0, groups, group, 0)

    pairs_w = npairs // _NW
    for j in range(pairs_w):
        bg = w * pairs_w + j
        pltpu.sync_copy(value_ref.at[bg], newbuf)
        pltpu.sync_copy(rowidx_ref.at[bg], idxbuf)
        pltpu.make_async_copy(newbuf, out_ref.at[idxbuf], sc_sem).start()
        pltpu.make_async_copy(newbuf, out_ref.at[idxbuf], sc_sem).wait()


def kernel(key, value, k_cache, v_cache, next_token_pos):
    B, G, L, H = k_cache.shape
    num = key.shape[2]
    BG = B * G

    key2 = key.reshape(BG, num, H)
    value2 = value.reshape(BG, num, H)
    kc2 = k_cache.reshape(BG, L, H)
    vc_flat = v_cache.reshape(BG * L, H)
    pos = jnp.asarray(next_token_pos, jnp.int32)
    pos1 = pos.reshape(1)
    rowidx = (jnp.arange(BG, dtype=jnp.int32)[:, None] * L + pos
              + jnp.arange(num, dtype=jnp.int32)[None, :])

    new_spec = pl.BlockSpec((BG, num, H), lambda i, p_ref: (0, 0, 0))
    win_spec = pl.BlockSpec((BG, num, H),
                            lambda i, p_ref: (0, p_ref[0] // num, 0))
    any_spec = pl.BlockSpec(memory_space=pl.ANY)
    grid_spec = pltpu.PrefetchScalarGridSpec(
        num_scalar_prefetch=1,
        grid=(1,),
        in_specs=[new_spec, any_spec],
        out_specs=[win_spec],
    )
    ko, = pl.pallas_call(
        _tc_k_body,
        grid_spec=grid_spec,
        out_shape=[jax.ShapeDtypeStruct((BG, L, H), k_cache.dtype)],
        input_output_aliases={2: 0},
    )(pos1, key2, kc2)

    mesh = plsc.VectorSubcoreMesh(core_axis_name="c", subcore_axis_name="s",
                                  num_cores=_NC, num_subcores=_NS)
    sc_call = functools.partial(
        pl.kernel,
        out_type=jax.ShapeDtypeStruct((BG * L, H), v_cache.dtype),
        mesh=mesh,
        scratch_types=[
            pltpu.VMEM((_NB, _CH, H), v_cache.dtype),
            pltpu.VMEM((num, H), v_cache.dtype),
            pltpu.VMEM((num,), jnp.int32),
            pltpu.SemaphoreType.DMA((_NB,)),
            pltpu.SemaphoreType.DMA((_NB,)),
            pltpu.SemaphoreType.DMA,
        ],
    )(_sc_v_body)
    vo = sc_call(value2, vc_flat, rowidx)

    return ko.reshape(B, G, L, H), vo.reshape(B, G, L, H)


# speedup vs baseline: 1.0156x; 1.0156x over previous
import functools

import jax
import jax.numpy as jnp
from jax import lax
from jax.experimental import pallas as pl
from jax.experimental.pallas import tpu as pltpu
from jax.experimental.pallas import tpu_sc as plsc

_NC = 2
_NS = 16
_NW = _NC * _NS
_CH = 64
_NB = 8


def _tc_k_body(pos_ref, key_ref, kc_ref, ko_ref):
    del pos_ref, kc_ref
    ko_ref[...] = key_ref[...]


def _sc_v_body(value_ref, vc_ref, rowidx_ref, out_ref,
               buf, newbuf, idxbuf, in_sems, out_sems, sc_sem):
    nrows = vc_ref.shape[0]
    npairs = value_ref.shape[0]
    num = value_ref.shape[1]
    w = lax.axis_index("s") * _NC + lax.axis_index("c")
    rows_w = nrows // _NW
    base = w * rows_w
    groups = rows_w // (_CH * _NB)

    for b in range(_NB):
        pltpu.make_async_copy(
            vc_ref.at[pl.ds(base + b * _CH, _CH)], buf.at[b],
            in_sems.at[b]).start()

    def group(gi, carry):
        g0 = base + gi * (_CH * _NB)
        for b in range(_NB):
            pltpu.make_async_copy(
                vc_ref.at[pl.ds(g0 + b * _CH, _CH)], buf.at[b],
                in_sems.at[b]).wait()
            pltpu.make_async_copy(
                buf.at[b], out_ref.at[pl.ds(g0 + b * _CH, _CH)],
                out_sems.at[b]).start()
        n0 = g0 + _NB * _CH

        @pl.when(gi + 1 < groups)
        def _():
            for b in range(_NB):
                pltpu.make_async_copy(
                    buf.at[b], out_ref.at[pl.ds(g0 + b * _CH, _CH)],
                    out_sems.at[b]).wait()
                pltpu.make_async_copy(
                    vc_ref.at[pl.ds(n0 + b * _CH, _CH)], buf.at[b],
                    in_sems.at[b]).start()

        @pl.when(gi + 1 >= groups)
        def _():
            for b in range(_NB):
                pltpu.make_async_copy(
                    buf.at[b], out_ref.at[pl.ds(g0 + b * _CH, _CH)],
                    out_sems.at[b]).wait()
        return carry

    lax.fori_loop(0, groups, group, 0)

    pairs_w = npairs // _NW
    for j in range(pairs_w):
        bg = w * pairs_w + j
        pltpu.sync_copy(value_ref.at[bg], newbuf)
        pltpu.sync_copy(rowidx_ref.at[bg], idxbuf)
        pltpu.make_async_copy(newbuf, out_ref.at[idxbuf], sc_sem).start()
        pltpu.make_async_copy(newbuf, out_ref.at[idxbuf], sc_sem).wait()


def kernel(key, value, k_cache, v_cache, next_token_pos):
    B, G, L, H = k_cache.shape
    num = key.shape[2]
    BG = B * G

    key2 = key.reshape(BG, num, H)
    value2 = value.reshape(BG, num, H)
    kc2 = k_cache.reshape(BG, L, H)
    vc_flat = v_cache.reshape(BG * L, H)
    pos = jnp.asarray(next_token_pos, jnp.int32)
    pos1 = pos.reshape(1)
    rowidx = (jnp.arange(BG, dtype=jnp.int32)[:, None] * L + pos
              + jnp.arange(num, dtype=jnp.int32)[None, :])

    new_spec = pl.BlockSpec((BG, num, H), lambda i, p_ref: (0, 0, 0))
    win_spec = pl.BlockSpec((BG, num, H),
                            lambda i, p_ref: (0, p_ref[0] // num, 0))
    any_spec = pl.BlockSpec(memory_space=pl.ANY)
    grid_spec = pltpu.PrefetchScalarGridSpec(
        num_scalar_prefetch=1,
        grid=(1,),
        in_specs=[new_spec, any_spec],
        out_specs=[win_spec],
    )
    ko, = pl.pallas_call(
        _tc_k_body,
        grid_spec=grid_spec,
        out_shape=[jax.ShapeDtypeStruct((BG, L, H), k_cache.dtype)],
        input_output_aliases={2: 0},
    )(pos1, key2, kc2)

    mesh = plsc.VectorSubcoreMesh(core_axis_name="c", subcore_axis_name="s",
                                  num_cores=_NC, num_subcores=_NS)
    sc_call = functools.partial(
        pl.kernel,
        out_type=jax.ShapeDtypeStruct((BG * L, H), v_cache.dtype),
        mesh=mesh,
        scratch_types=[
            pltpu.VMEM((_NB, _CH, H), v_cache.dtype),
            pltpu.VMEM((num, H), v_cache.dtype),
            pltpu.VMEM((num,), jnp.int32),
            pltpu.SemaphoreType.DMA((_NB,)),
            pltpu.SemaphoreType.DMA((_NB,)),
            pltpu.SemaphoreType.DMA,
        ],
    )(_sc_v_body)
    vo = sc_call(value2, vc_flat, rowidx)

    return ko.reshape(B, G, L, H), vo.reshape(B, G, L, H)


# speedup vs baseline: 1.0431x; 1.0270x over previous
import functools

import jax
import jax.numpy as jnp
from jax import lax
from jax.experimental import pallas as pl
from jax.experimental.pallas import tpu as pltpu
from jax.experimental.pallas import tpu_sc as plsc

_NC = 1
_NS = 16
_NW = _NC * _NS


def _sc_scatter_body(key_ref, value_ref, rowidx_ref, ko_ref, vo_ref,
                     newbuf, idxbuf, sem):
    npairs = key_ref.shape[0]
    w = lax.axis_index("s") * _NC + lax.axis_index("c")
    pairs_w = npairs // _NW
    for j in range(pairs_w):
        bg = w * pairs_w + j
        pltpu.sync_copy(rowidx_ref.at[bg], idxbuf)
        pltpu.sync_copy(key_ref.at[bg], newbuf)
        pltpu.make_async_copy(newbuf, ko_ref.at[idxbuf], sem).start()
        pltpu.make_async_copy(newbuf, ko_ref.at[idxbuf], sem).wait()
        pltpu.sync_copy(value_ref.at[bg], newbuf)
        pltpu.make_async_copy(newbuf, vo_ref.at[idxbuf], sem).start()
        pltpu.make_async_copy(newbuf, vo_ref.at[idxbuf], sem).wait()


def kernel(key, value, k_cache, v_cache, next_token_pos):
    B, G, L, H = k_cache.shape
    num = key.shape[2]
    BG = B * G

    key2 = key.reshape(BG, num, H)
    value2 = value.reshape(BG, num, H)
    pos = jnp.asarray(next_token_pos, jnp.int32)
    rowidx = (jnp.arange(BG, dtype=jnp.int32)[:, None] * L + pos
              + jnp.arange(num, dtype=jnp.int32)[None, :])

    ko_ref = jax.new_ref(k_cache.reshape(BG * L, H))
    vo_ref = jax.new_ref(v_cache.reshape(BG * L, H))

    mesh = plsc.VectorSubcoreMesh(core_axis_name="c", subcore_axis_name="s",
                                  num_cores=_NC, num_subcores=_NS)
    sc_scatter = functools.partial(
        pl.kernel,
        out_type=(),
        mesh=mesh,
        scratch_types=[
            pltpu.VMEM((num, H), k_cache.dtype),
            pltpu.VMEM((num,), jnp.int32),
            pltpu.SemaphoreType.DMA,
        ],
    )(_sc_scatter_body)
    sc_scatter(key2, value2, rowidx, ko_ref, vo_ref)

    return (ko_ref[...].reshape(B, G, L, H),
            vo_ref[...].reshape(B, G, L, H))


# speedup vs baseline: 1.0842x; 1.0394x over previous
import functools

import jax
import jax.numpy as jnp
from jax import lax
from jax.experimental import pallas as pl
from jax.experimental.pallas import tpu as pltpu
from jax.experimental.pallas import tpu_sc as plsc

_NC = 1
_NS = 16
_NW = _NC * _NS


def _sc_scatter_body(key_ref, value_ref, rowidx_ref, ko_ref, vo_ref,
                     newbuf, idxbuf, gsem, ssem):
    npairs = key_ref.shape[0]
    w = lax.axis_index("s") * _NC + lax.axis_index("c")
    pairs_w = npairs // _NW
    stages = []
    for j in range(pairs_w):
        bg = w * pairs_w + j
        stages.append(pltpu.make_async_copy(
            key_ref.at[bg], newbuf.at[2 * j], gsem))
        stages.append(pltpu.make_async_copy(
            value_ref.at[bg], newbuf.at[2 * j + 1], gsem))
        stages.append(pltpu.make_async_copy(
            rowidx_ref.at[bg], idxbuf.at[j], gsem))
    for d in stages:
        d.start()
    for d in stages:
        d.wait()
    scats = []
    for j in range(pairs_w):
        scats.append(pltpu.make_async_copy(
            newbuf.at[2 * j], ko_ref.at[idxbuf.at[j]], ssem))
        scats.append(pltpu.make_async_copy(
            newbuf.at[2 * j + 1], vo_ref.at[idxbuf.at[j]], ssem))
    for d in scats:
        d.start()
    for d in scats:
        d.wait()


def kernel(key, value, k_cache, v_cache, next_token_pos):
    B, G, L, H = k_cache.shape
    num = key.shape[2]
    BG = B * G

    key2 = key.reshape(BG, num, H)
    value2 = value.reshape(BG, num, H)
    pos = jnp.asarray(next_token_pos, jnp.int32)
    rowidx = (jnp.arange(BG, dtype=jnp.int32)[:, None] * L + pos
              + jnp.arange(num, dtype=jnp.int32)[None, :])

    ko_ref = jax.new_ref(k_cache.reshape(BG * L, H))
    vo_ref = jax.new_ref(v_cache.reshape(BG * L, H))

    mesh = plsc.VectorSubcoreMesh(core_axis_name="c", subcore_axis_name="s",
                                  num_cores=_NC, num_subcores=_NS)
    sc_scatter = functools.partial(
        pl.kernel,
        out_type=(),
        mesh=mesh,
        scratch_types=[
            pltpu.VMEM((2 * (BG // _NW), num, H), k_cache.dtype),
            pltpu.VMEM((BG // _NW, num), jnp.int32),
            pltpu.SemaphoreType.DMA,
            pltpu.SemaphoreType.DMA,
        ],
    )(_sc_scatter_body)
    sc_scatter(key2, value2, rowidx, ko_ref, vo_ref)

    return (ko_ref[...].reshape(B, G, L, H),
            vo_ref[...].reshape(B, G, L, H))
